# baseline (speedup 1.0000x reference)
import functools

import jax
import jax.numpy as jnp
from jax import lax
from jax.experimental import pallas as pl
from jax.experimental.pallas import tpu as pltpu
from jax.experimental.pallas import tpu_sc as plsc

MAXLEN = 200
EMBED = 64
LANES = 16
NC, NS = 2, 16
NW = NC * NS
BBLK = 128
NBH = 4096 // BBLK
EH, EL = EMBED // 8, 8


def _sc_embed(xt, tok2, pos_table):
    n_blocks = xt.shape[0]
    bpw = n_blocks // NW
    steps = bpw // 2
    mesh = plsc.VectorSubcoreMesh(core_axis_name="c", subcore_axis_name="s")

    @functools.partial(
        pl.kernel,
        out_type=jax.ShapeDtypeStruct((MAXLEN, EH, NBH, EL, BBLK), jnp.float32),
        mesh=mesh,
        scratch_types=[
            pltpu.VMEM((bpw, BBLK), jnp.int32),
            pltpu.VMEM((BBLK, EMBED), jnp.float32),
            pltpu.VMEM((BBLK, EMBED), jnp.float32),
            pltpu.VMEM((EH, EL, BBLK), jnp.float32),
            pltpu.VMEM((EH, EL, BBLK), jnp.float32),
            pltpu.VMEM((MAXLEN, EMBED), jnp.float32),
            pltpu.SemaphoreType.DMA,
            pltpu.SemaphoreType.DMA,
            pltpu.SemaphoreType.DMA,
            pltpu.SemaphoreType.DMA,
        ],
        compiler_params=pltpu.CompilerParams(
            use_tc_tiling_on_sc=False, needs_layout_passes=False),
    )
    def k(xt_hbm, tok_hbm, pos_hbm, out_hbm, idx_v, rows0, rows1, ob0, ob1,
          pos_v, gsem0, gsem1, ssem0, ssem1):
        wid = lax.axis_index("s") * NC + lax.axis_index("c")
        base = wid * bpw
        pltpu.sync_copy(pos_hbm, pos_v)
        pltpu.sync_copy(xt_hbm.at[pl.ds(base, bpw)], idx_v)

        iota = lax.iota(jnp.int32, LANES)
        row_ids = [iota + jj * LANES for jj in range(BBLK // LANES)]

        def transpose_add(rows, ob, p):
            p16 = jnp.full((LANES,), 0, jnp.int32) + p

            def ebody(e, carry):
                e16 = jnp.full((LANES,), 0, jnp.int32) + e
                pv = plsc.load_gather(pos_v, [p16, e16])
                eh = e // EL
                el = e % EL
                for jj in range(BBLK // LANES):
                    col = plsc.load_gather(rows, [row_ids[jj], e16])
                    ob[eh, el, pl.ds(jj * LANES, LANES)] = col + pv
                return carry

            lax.fori_loop(0, EMBED, ebody, 0)

        def scat(ob, p, bh, sem):
            for eh in range(EH):
                pltpu.async_copy(ob.at[eh], out_hbm.at[p, eh, bh], sem)

        def scat_wait(ob, p, bh, sem):
            for eh in range(EH):
                pltpu.make_async_copy(ob.at[eh], out_hbm.at[p, eh, bh], sem).wait()

        def blk(t):
            return (base + t) // NBH, (base + t) % NBH

        pltpu.async_copy(tok_hbm.at[idx_v.at[0]], rows0, gsem0)

        def step(g, carry):
            c0 = 2 * g
            p0, bh0 = blk(c0)
            p1, bh1 = blk(c0 + 1)
            pm1, bhm1 = blk(c0 - 1)
            pm2, bhm2 = blk(c0 - 2)

            pltpu.async_copy(tok_hbm.at[idx_v.at[c0 + 1]], rows1, gsem1)
            pltpu.make_async_copy(tok_hbm.at[idx_v.at[c0]], rows0, gsem0).wait()

            @pl.when(g > 0)
            def _():
                scat_wait(ob0, pm2, bhm2, ssem0)

            transpose_add(rows0, ob0, p0)
            scat(ob0, p0, bh0, ssem0)

            @pl.when(g < steps - 1)
            def _():
                pltpu.async_copy(tok_hbm.at[idx_v.at[c0 + 2]], rows0, gsem0)

            pltpu.make_async_copy(tok_hbm.at[idx_v.at[c0 + 1]], rows1, gsem1).wait()

            @pl.when(g > 0)
            def _():
                scat_wait(ob1, pm1, bhm1, ssem1)

            transpose_add(rows1, ob1, p1)
            scat(ob1, p1, bh1, ssem1)
            return carry

        lax.fori_loop(0, steps, step, 0)
        pL0, bhL0 = blk(bpw - 2)
        pL1, bhL1 = blk(bpw - 1)
        scat_wait(ob0, pL0, bhL0, ssem0)
        scat_wait(ob1, pL1, bhL1, ssem1)

    return k(xt, tok2, pos_table)


def kernel(x, token_table, pos_table):
    batch, seqlen = x.shape
    xt = jnp.transpose(x.astype(jnp.int32)).reshape(seqlen * (batch // BBLK), BBLK)
    tok_pairs = lax.optimization_barrier(
        jnp.reshape(token_table, (token_table.shape[0] // 2, 2 * EMBED)))
    tok2 = jnp.reshape(tok_pairs, (token_table.shape[0], EMBED))
    out5 = _sc_embed(xt, tok2, pos_table)
    out = jnp.transpose(out5, (2, 4, 0, 1, 3)).reshape(batch, seqlen, EMBED)
    return out


# speedup vs baseline: 1.7196x; 1.7196x over previous
import functools

import jax
import jax.numpy as jnp
from jax import lax
from jax.experimental import pallas as pl
from jax.experimental.pallas import tpu as pltpu
from jax.experimental.pallas import tpu_sc as plsc

MAXLEN = 200
EMBED = 64
LANES = 16
NC, NS = 2, 16
NW = NC * NS
BBLK = 128
NBH = 4096 // BBLK
EH, EL = EMBED // 8, 8
OBW = BBLK + 1


def _sc_embed(xt, tok2, pos_table):
    n_blocks = xt.shape[0]
    bpw = n_blocks // NW
    steps = bpw // 2
    mesh = plsc.VectorSubcoreMesh(core_axis_name="c", subcore_axis_name="s")

    @functools.partial(
        pl.kernel,
        out_type=jax.ShapeDtypeStruct((MAXLEN, EH, NBH, EL, BBLK), jnp.float32),
        mesh=mesh,
        scratch_types=[
            pltpu.VMEM((bpw, BBLK), jnp.int32),
            pltpu.VMEM((BBLK, EMBED), jnp.float32),
            pltpu.VMEM((BBLK, EMBED), jnp.float32),
            pltpu.VMEM((EMBED, OBW), jnp.float32),
            pltpu.VMEM((EMBED, OBW), jnp.float32),
            pltpu.VMEM((MAXLEN, EMBED), jnp.float32),
            pltpu.SemaphoreType.DMA,
            pltpu.SemaphoreType.DMA,
            pltpu.SemaphoreType.DMA,
            pltpu.SemaphoreType.DMA,
        ],
        compiler_params=pltpu.CompilerParams(
            use_tc_tiling_on_sc=False, needs_layout_passes=False),
    )
    def k(xt_hbm, tok_hbm, pos_hbm, out_hbm, idx_v, rows0, rows1, ob0, ob1,
          pos_v, gsem0, gsem1, ssem0, ssem1):
        wid = lax.axis_index("s") * NC + lax.axis_index("c")
        base = wid * bpw
        pltpu.sync_copy(pos_hbm, pos_v)
        pltpu.sync_copy(xt_hbm.at[pl.ds(base, bpw)], idx_v)

        iota = lax.iota(jnp.int32, LANES)
        e_ids = [iota + k * LANES for k in range(EMBED // LANES)]

        def transpose_add(rows, ob, p):
            pvs = [pos_v[p, pl.ds(k * LANES, LANES)]
                   for k in range(EMBED // LANES)]

            def tbody(t, carry):
                t16 = jnp.full((LANES,), 0, jnp.int32) + t
                for k in range(EMBED // LANES):
                    v = rows[t, pl.ds(k * LANES, LANES)] + pvs[k]
                    plsc.store_scatter(ob, [e_ids[k], t16], v)
                return carry

            lax.fori_loop(0, BBLK, tbody, 0)

        def scat(ob, p, bh, sem):
            for eh in range(EH):
                pltpu.async_copy(ob.at[pl.ds(eh * EL, EL), pl.ds(0, BBLK)],
                                 out_hbm.at[p, eh, bh], sem)

        def scat_wait(ob, p, bh, sem):
            for eh in range(EH):
                pltpu.make_async_copy(
                    ob.at[pl.ds(eh * EL, EL), pl.ds(0, BBLK)],
                    out_hbm.at[p, eh, bh], sem).wait()

        def blk(t):
            return (base + t) // NBH, (base + t) % NBH

        pltpu.async_copy(tok_hbm.at[idx_v.at[0]], rows0, gsem0)

        def step(g, carry):
            c0 = 2 * g
            p0, bh0 = blk(c0)
            p1, bh1 = blk(c0 + 1)
            pm1, bhm1 = blk(c0 - 1)
            pm2, bhm2 = blk(c0 - 2)

            pltpu.async_copy(tok_hbm.at[idx_v.at[c0 + 1]], rows1, gsem1)
            pltpu.make_async_copy(tok_hbm.at[idx_v.at[c0]], rows0, gsem0).wait()

            @pl.when(g > 0)
            def _():
                scat_wait(ob0, pm2, bhm2, ssem0)

            transpose_add(rows0, ob0, p0)
            scat(ob0, p0, bh0, ssem0)

            @pl.when(g < steps - 1)
            def _():
                pltpu.async_copy(tok_hbm.at[idx_v.at[c0 + 2]], rows0, gsem0)

            pltpu.make_async_copy(tok_hbm.at[idx_v.at[c0 + 1]], rows1, gsem1).wait()

            @pl.when(g > 0)
            def _():
                scat_wait(ob1, pm1, bhm1, ssem1)

            transpose_add(rows1, ob1, p1)
            scat(ob1, p1, bh1, ssem1)
            return carry

        lax.fori_loop(0, steps, step, 0)
        pL0, bhL0 = blk(bpw - 2)
        pL1, bhL1 = blk(bpw - 1)
        scat_wait(ob0, pL0, bhL0, ssem0)
        scat_wait(ob1, pL1, bhL1, ssem1)

    return k(xt, tok2, pos_table)


def kernel(x, token_table, pos_table):
    batch, seqlen = x.shape
    xt = jnp.transpose(x.astype(jnp.int32)).reshape(seqlen * (batch // BBLK), BBLK)
    tok_pairs = lax.optimization_barrier(
        jnp.reshape(token_table, (token_table.shape[0] // 2, 2 * EMBED)))
    tok2 = jnp.reshape(tok_pairs, (token_table.shape[0], EMBED))
    out5 = _sc_embed(xt, tok2, pos_table)
    out = jnp.transpose(out5, (2, 4, 0, 1, 3)).reshape(batch, seqlen, EMBED)
    return out


# speedup vs baseline: 1.7531x; 1.0195x over previous
import functools

import jax
import jax.numpy as jnp
from jax import lax
from jax.experimental import pallas as pl
from jax.experimental.pallas import tpu as pltpu
from jax.experimental.pallas import tpu_sc as plsc

MAXLEN = 200
EMBED = 64
LANES = 16
NC, NS = 2, 16
NW = NC * NS
BBLK = 128
NBH = 4096 // BBLK
EH, EL = EMBED // 8, 8
OBW = BBLK + 1


def _sc_embed(xt, tok2, pos_table):
    n_blocks = xt.shape[0]
    bpw = n_blocks // NW
    steps = bpw // 2
    mesh = plsc.VectorSubcoreMesh(core_axis_name="c", subcore_axis_name="s")

    @functools.partial(
        pl.kernel,
        out_type=jax.ShapeDtypeStruct((MAXLEN, EH, NBH, EL, BBLK), jnp.float32),
        mesh=mesh,
        scratch_types=[
            pltpu.VMEM((bpw, BBLK), jnp.int32),
            pltpu.VMEM((BBLK, EMBED), jnp.float32),
            pltpu.VMEM((BBLK, EMBED), jnp.float32),
            pltpu.VMEM((EMBED, OBW), jnp.float32),
            pltpu.VMEM((EMBED, OBW), jnp.float32),
            pltpu.VMEM((MAXLEN, EMBED), jnp.float32),
            pltpu.SemaphoreType.DMA,
            pltpu.SemaphoreType.DMA,
            pltpu.SemaphoreType.DMA,
            pltpu.SemaphoreType.DMA,
        ],
        compiler_params=pltpu.CompilerParams(
            use_tc_tiling_on_sc=False, needs_layout_passes=False),
    )
    def k(xt_hbm, tok_hbm, pos_hbm, out_hbm, idx_v, rows0, rows1, ob0, ob1,
          pos_v, gsem0, gsem1, ssem0, ssem1):
        wid = lax.axis_index("s") * NC + lax.axis_index("c")
        base = wid * bpw
        pltpu.sync_copy(pos_hbm, pos_v)
        pltpu.sync_copy(xt_hbm.at[pl.ds(base, bpw)], idx_v)

        iota = lax.iota(jnp.int32, LANES)
        e_ids = [iota + k * LANES for k in range(EMBED // LANES)]

        def transpose_add(rows, ob, p):
            pvs = [pos_v[p, pl.ds(k * LANES, LANES)]
                   for k in range(EMBED // LANES)]

            def tbody(tq, carry):
                for dt in range(4):
                    t = tq * 4 + dt
                    t16 = jnp.full((LANES,), 0, jnp.int32) + t
                    for k in range(EMBED // LANES):
                        v = rows[t, pl.ds(k * LANES, LANES)] + pvs[k]
                        plsc.store_scatter(ob, [e_ids[k], t16], v)
                return carry

            lax.fori_loop(0, BBLK // 4, tbody, 0)

        def scat(ob, p, bh, sem):
            for eh in range(EH):
                pltpu.async_copy(ob.at[pl.ds(eh * EL, EL), pl.ds(0, BBLK)],
                                 out_hbm.at[p, eh, bh], sem)

        def scat_wait(ob, p, bh, sem):
            for eh in range(EH):
                pltpu.make_async_copy(
                    ob.at[pl.ds(eh * EL, EL), pl.ds(0, BBLK)],
                    out_hbm.at[p, eh, bh], sem).wait()

        def blk(t):
            return (base + t) // NBH, (base + t) % NBH

        pltpu.async_copy(tok_hbm.at[idx_v.at[0]], rows0, gsem0)

        def step(g, carry):
            c0 = 2 * g
            p0, bh0 = blk(c0)
            p1, bh1 = blk(c0 + 1)
            pm1, bhm1 = blk(c0 - 1)
            pm2, bhm2 = blk(c0 - 2)

            pltpu.async_copy(tok_hbm.at[idx_v.at[c0 + 1]], rows1, gsem1)
            pltpu.make_async_copy(tok_hbm.at[idx_v.at[c0]], rows0, gsem0).wait()

            @pl.when(g > 0)
            def _():
                scat_wait(ob0, pm2, bhm2, ssem0)

            transpose_add(rows0, ob0, p0)
            scat(ob0, p0, bh0, ssem0)

            @pl.when(g < steps - 1)
            def _():
                pltpu.async_copy(tok_hbm.at[idx_v.at[c0 + 2]], rows0, gsem0)

            pltpu.make_async_copy(tok_hbm.at[idx_v.at[c0 + 1]], rows1, gsem1).wait()

            @pl.when(g > 0)
            def _():
                scat_wait(ob1, pm1, bhm1, ssem1)

            transpose_add(rows1, ob1, p1)
            scat(ob1, p1, bh1, ssem1)
            return carry

        lax.fori_loop(0, steps, step, 0)
        pL0, bhL0 = blk(bpw - 2)
        pL1, bhL1 = blk(bpw - 1)
        scat_wait(ob0, pL0, bhL0, ssem0)
        scat_wait(ob1, pL1, bhL1, ssem1)

    return k(xt, tok2, pos_table)


def kernel(x, token_table, pos_table):
    batch, seqlen = x.shape
    xt = jnp.transpose(x.astype(jnp.int32)).reshape(seqlen * (batch // BBLK), BBLK)
    tok_pairs = lax.optimization_barrier(
        jnp.reshape(token_table, (token_table.shape[0] // 2, 2 * EMBED)))
    tok2 = jnp.reshape(tok_pairs, (token_table.shape[0], EMBED))
    out5 = _sc_embed(xt, tok2, pos_table)
    out = jnp.transpose(out5, (2, 4, 0, 1, 3)).reshape(batch, seqlen, EMBED)
    return out


# speedup vs baseline: 2.5348x; 1.4459x over previous
import functools

import jax
import jax.numpy as jnp
from jax import lax
from jax.experimental import pallas as pl
from jax.experimental.pallas import tpu as pltpu
from jax.experimental.pallas import tpu_sc as plsc

MAXLEN = 200
EMBED = 64
LANES = 16
NC, NS = 2, 16
NW = NC * NS
BBLK = 128
NBH = 4096 // BBLK
EH, EL = EMBED // 8, 8
OBW = BBLK + 1


def _sc_embed(xt, tok2, pos_table):
    n_blocks = xt.shape[0]
    bpw = n_blocks // NW
    steps = bpw // 2
    mesh = plsc.VectorSubcoreMesh(core_axis_name="c", subcore_axis_name="s")

    @functools.partial(
        pl.kernel,
        out_type=jax.ShapeDtypeStruct((MAXLEN, EH, NBH, EL, BBLK), jnp.float32),
        mesh=mesh,
        scratch_types=[
            pltpu.VMEM((bpw, BBLK), jnp.int32),
            pltpu.VMEM((BBLK, EMBED), jnp.float32),
            pltpu.VMEM((BBLK, EMBED), jnp.float32),
            pltpu.VMEM((EMBED, OBW), jnp.float32),
            pltpu.VMEM((EMBED, OBW), jnp.float32),
            pltpu.VMEM((MAXLEN, EMBED), jnp.float32),
            pltpu.SemaphoreType.DMA,
            pltpu.SemaphoreType.DMA,
            pltpu.SemaphoreType.DMA,
            pltpu.SemaphoreType.DMA,
        ],
        compiler_params=pltpu.CompilerParams(
            use_tc_tiling_on_sc=False, needs_layout_passes=False),
    )
    def k(xt_hbm, tok_hbm, pos_hbm, out_hbm, idx_v, rows0, rows1, ob0, ob1,
          pos_v, gsem0, gsem1, ssem0, ssem1):
        wid = lax.axis_index("s") * NC + lax.axis_index("c")
        base = wid * bpw
        pltpu.sync_copy(pos_hbm, pos_v)
        pltpu.sync_copy(xt_hbm.at[pl.ds(base, bpw)], idx_v)

        iota = lax.iota(jnp.int32, LANES)
        e_ids = [iota + k * LANES for k in range(EMBED // LANES)]

        def transpose_add(rows, ob, p):
            pvs = [pos_v[p, pl.ds(k * LANES, LANES)]
                   for k in range(EMBED // LANES)]

            @functools.partial(plsc.parallel_loop, 0, BBLK, unroll=4)
            def tbody(t):
                t16 = jnp.full((LANES,), 0, jnp.int32) + t
                for k in range(EMBED // LANES):
                    v = rows[t, pl.ds(k * LANES, LANES)] + pvs[k]
                    plsc.store_scatter(ob, [e_ids[k], t16], v)

        def scat(ob, p, bh, sem):
            for eh in range(EH):
                pltpu.async_copy(ob.at[pl.ds(eh * EL, EL), pl.ds(0, BBLK)],
                                 out_hbm.at[p, eh, bh], sem)

        def scat_wait(ob, p, bh, sem):
            for eh in range(EH):
                pltpu.make_async_copy(
                    ob.at[pl.ds(eh * EL, EL), pl.ds(0, BBLK)],
                    out_hbm.at[p, eh, bh], sem).wait()

        def blk(t):
            return (base + t) // NBH, (base + t) % NBH

        pltpu.async_copy(tok_hbm.at[idx_v.at[0]], rows0, gsem0)

        def step(g, carry):
            c0 = 2 * g
            p0, bh0 = blk(c0)
            p1, bh1 = blk(c0 + 1)
            pm1, bhm1 = blk(c0 - 1)
            pm2, bhm2 = blk(c0 - 2)

            pltpu.async_copy(tok_hbm.at[idx_v.at[c0 + 1]], rows1, gsem1)
            pltpu.make_async_copy(tok_hbm.at[idx_v.at[c0]], rows0, gsem0).wait()

            @pl.when(g > 0)
            def _():
                scat_wait(ob0, pm2, bhm2, ssem0)

            transpose_add(rows0, ob0, p0)
            scat(ob0, p0, bh0, ssem0)

            @pl.when(g < steps - 1)
            def _():
                pltpu.async_copy(tok_hbm.at[idx_v.at[c0 + 2]], rows0, gsem0)

            pltpu.make_async_copy(tok_hbm.at[idx_v.at[c0 + 1]], rows1, gsem1).wait()

            @pl.when(g > 0)
            def _():
                scat_wait(ob1, pm1, bhm1, ssem1)

            transpose_add(rows1, ob1, p1)
            scat(ob1, p1, bh1, ssem1)
            return carry

        lax.fori_loop(0, steps, step, 0)
        pL0, bhL0 = blk(bpw - 2)
        pL1, bhL1 = blk(bpw - 1)
        scat_wait(ob0, pL0, bhL0, ssem0)
        scat_wait(ob1, pL1, bhL1, ssem1)

    return k(xt, tok2, pos_table)


def kernel(x, token_table, pos_table):
    batch, seqlen = x.shape
    xt = jnp.transpose(x.astype(jnp.int32)).reshape(seqlen * (batch // BBLK), BBLK)
    tok_pairs = lax.optimization_barrier(
        jnp.reshape(token_table, (token_table.shape[0] // 2, 2 * EMBED)))
    tok2 = jnp.reshape(tok_pairs, (token_table.shape[0], EMBED))
    out5 = _sc_embed(xt, tok2, pos_table)
    out = jnp.transpose(out5, (2, 4, 0, 1, 3)).reshape(batch, seqlen, EMBED)
    return out
